# baseline (speedup 1.0000x reference)
import functools

import jax
import jax.numpy as jnp
from jax import lax
from jax.experimental import pallas as pl
from jax.experimental.pallas import tpu as pltpu
from jax.experimental.pallas import tpu_sc as plsc

BATCH = 16384
EMBED = 64
VOCAB_HALF = 50000
NUM_TABLES = 3
NW = 32
BPW = BATCH // NW
CHUNK = 256
NCHUNK = BPW // CHUNK

_mesh = plsc.VectorSubcoreMesh(core_axis_name="c", subcore_axis_name="s")


@functools.partial(
    pl.kernel,
    mesh=_mesh,
    out_type=(
        jax.ShapeDtypeStruct((BATCH, 2 * EMBED), jnp.float32),
        jax.ShapeDtypeStruct((BATCH, 2 * EMBED), jnp.float32),
        jax.ShapeDtypeStruct((BATCH, 2 * EMBED), jnp.float32),
    ),
    scratch_types=[
        pltpu.VMEM((NUM_TABLES * BPW,), jnp.int32),
        pltpu.VMEM((CHUNK, 2 * EMBED), jnp.float32),
        pltpu.VMEM((CHUNK, 2 * EMBED), jnp.float32),
        pltpu.VMEM((CHUNK, 2 * EMBED), jnp.float32),
        pltpu.SemaphoreType.DMA,
        pltpu.SemaphoreType.DMA,
        pltpu.SemaphoreType.DMA,
    ],
)
def _emb_kernel(gidx_hbm, t0_hbm, t1_hbm, t2_hbm, o0_hbm, o1_hbm, o2_hbm,
                gidx, b0, b1, b2, s0, s1, s2):
    wid = lax.axis_index("s") * 2 + lax.axis_index("c")
    base = wid * BPW
    pltpu.sync_copy(gidx_hbm.at[pl.ds(base, BPW)], gidx.at[pl.ds(0, BPW)])
    pltpu.sync_copy(gidx_hbm.at[pl.ds(BATCH + base, BPW)],
                    gidx.at[pl.ds(BPW, BPW)])
    pltpu.sync_copy(gidx_hbm.at[pl.ds(2 * BATCH + base, BPW)],
                    gidx.at[pl.ds(2 * BPW, BPW)])

    tables = (t0_hbm, t1_hbm, t2_hbm)
    outs = (o0_hbm, o1_hbm, o2_hbm)
    bufs = (b0, b1, b2)
    sems = (s0, s1, s2)
    jobs = [(t, c) for t in range(NUM_TABLES) for c in range(NCHUNK)]

    def fire(j):
        t, c = jobs[j]
        idx_slice = gidx.at[pl.ds(t * BPW + c * CHUNK, CHUNK)]
        return pltpu.async_copy(tables[t].at[idx_slice], bufs[j % 3],
                                sems[j % 3])

    inflight = [fire(0), fire(1), fire(2)]
    for j in range(len(jobs)):
        t, c = jobs[j]
        inflight[j % 3].wait()
        pltpu.sync_copy(bufs[j % 3],
                        outs[t].at[pl.ds(base + c * CHUNK, CHUNK)])
        if j + 3 < len(jobs):
            inflight[j % 3] = fire(j + 3)


def kernel(x, emb_mi, emb_mo, emb_mtext):
    t0 = emb_mi.reshape(VOCAB_HALF, 2 * EMBED)
    t1 = emb_mo.reshape(VOCAB_HALF, 2 * EMBED)
    t2 = emb_mtext.reshape(VOCAB_HALF, 2 * EMBED)
    gidx = jnp.transpose(x >> 1).reshape(NUM_TABLES * BATCH)
    o0, o1, o2 = _emb_kernel(gidx, t0, t1, t2)
    parts = []
    for t, o in enumerate((o0, o1, o2)):
        odd = (x[:, t] & 1)[:, None] == 1
        parts.append(jnp.where(odd, o[:, EMBED:], o[:, :EMBED]))
    return jnp.concatenate(parts, axis=1)


# speedup vs baseline: 1.2606x; 1.2606x over previous
import functools

import jax
import jax.numpy as jnp
from jax import lax
from jax.experimental import pallas as pl
from jax.experimental.pallas import tpu as pltpu
from jax.experimental.pallas import tpu_sc as plsc

BATCH = 16384
VOCAB = 100000
EMBED = 64
NUM_TABLES = 3
NROWS = NUM_TABLES * EMBED
NW = 32
RPW = NROWS // NW
HALF = BATCH // 2

_mesh = plsc.VectorSubcoreMesh(core_axis_name="c", subcore_axis_name="s")


@functools.partial(
    pl.kernel,
    mesh=_mesh,
    compiler_params=pltpu.CompilerParams(needs_layout_passes=False),
    out_type=jax.ShapeDtypeStruct((NROWS * BATCH,), jnp.float32),
    scratch_types=[
        pltpu.VMEM((VOCAB,), jnp.float32),
        pltpu.VMEM((HALF,), jnp.int32),
        pltpu.VMEM((HALF,), jnp.float32),
    ],
)
def _emb_kernel(xT_hbm, t0_hbm, t1_hbm, t2_hbm, out_hbm,
                row_buf, idx_buf, out_buf):
    wid = lax.axis_index("s") * 2 + lax.axis_index("c")
    for j in range(RPW):
        g = wid * RPW + j
        t = g // EMBED
        f = g % EMBED
        off = f * VOCAB

        @pl.when(t == 0)
        def _s0():
            pltpu.sync_copy(t0_hbm.at[pl.ds(off, VOCAB)], row_buf)

        @pl.when(t == 1)
        def _s1():
            pltpu.sync_copy(t1_hbm.at[pl.ds(off, VOCAB)], row_buf)

        @pl.when(t == 2)
        def _s2():
            pltpu.sync_copy(t2_hbm.at[pl.ds(off, VOCAB)], row_buf)

        for h in range(2):
            pltpu.sync_copy(xT_hbm.at[pl.ds(t * BATCH + h * HALF, HALF)],
                            idx_buf)

            @plsc.parallel_loop(0, HALF, 16, unroll=8)
            def _gather(i):
                v = idx_buf[pl.ds(i, 16)]
                out_buf[pl.ds(i, 16)] = plsc.load_gather(row_buf, [v])

            pltpu.sync_copy(out_buf,
                            out_hbm.at[pl.ds(g * BATCH + h * HALF, HALF)])


def kernel(x, emb_mi, emb_mo, emb_mtext):
    xT = jnp.transpose(x).reshape(NUM_TABLES * BATCH)
    t0 = jnp.transpose(emb_mi).reshape(EMBED * VOCAB)
    t1 = jnp.transpose(emb_mo).reshape(EMBED * VOCAB)
    t2 = jnp.transpose(emb_mtext).reshape(EMBED * VOCAB)
    out_flat = _emb_kernel(xT, t0, t1, t2)
    return jnp.transpose(out_flat.reshape(NROWS, BATCH))
